# Initial kernel scaffold; baseline (speedup 1.0000x reference)
import functools

import jax
import jax.numpy as jnp
from jax import lax
from jax.experimental import pallas as pl
from jax.experimental.pallas import tpu as pltpu
from jax.experimental.pallas import tpu_sc as plsc

NC = 2
NS = 16
NW = NC * NS

D = 32
L = 128
GROUP = 8
CHUNK = L * GROUP


def _build(n_tokens):
  assert n_tokens % (NW * CHUNK) == 0
  tok_rows_per_w = n_tokens // (NW * L)
  n_groups = tok_rows_per_w // GROUP
  tok_per_w = n_tokens // NW

  mesh = plsc.VectorSubcoreMesh(
      core_axis_name="c", subcore_axis_name="s",
      num_cores=NC, num_subcores=NS)

  @functools.partial(
      pl.kernel,
      out_type=jax.ShapeDtypeStruct((n_tokens, D), jnp.float32),
      mesh=mesh,
      scratch_types=[
          pltpu.VMEM((tok_rows_per_w, L), jnp.int32),
          pltpu.VMEM((CHUNK, D), jnp.float32),
          pltpu.SemaphoreType.DMA,
      ],
  )
  def embed(tok_hbm, w_hbm, out_hbm, idx_v, rows_v, gsem):
    wid = lax.axis_index("s") * NC + lax.axis_index("c")
    pltpu.sync_copy(tok_hbm.at[pl.ds(wid * tok_rows_per_w, tok_rows_per_w)],
                    idx_v)
    out_base = wid * tok_per_w

    @pl.loop(0, n_groups)
    def group_loop(g):
      cps = []
      for j in range(GROUP):
        cps.append(pltpu.async_copy(
            w_hbm.at[idx_v.at[g * GROUP + j]],
            rows_v.at[pl.ds(j * L, L)], gsem))
      for cp in cps:
        cp.wait()
      pltpu.sync_copy(rows_v, out_hbm.at[pl.ds(out_base + g * CHUNK, CHUNK)])

  return embed


def kernel(token_ids, weight):
  b, s = token_ids.shape
  n = b * s
  tok = token_ids.reshape(n // L, L).astype(jnp.int32)
  out = _build(n)(tok, weight)
  return out.reshape(b, s, D)


# speedup vs baseline: 1.4761x; 1.4761x over previous
import functools

import jax
import jax.numpy as jnp
from jax import lax
from jax.experimental import pallas as pl
from jax.experimental.pallas import tpu as pltpu
from jax.experimental.pallas import tpu_sc as plsc

NC = 2
NS = 16
NW = NC * NS

D = 32
L = 128
GROUP = 8
CHUNK = L * GROUP


def _build(n_tokens):
  assert n_tokens % (NW * CHUNK) == 0
  tok_rows_per_w = n_tokens // (NW * L)
  n_groups = tok_rows_per_w // GROUP
  tok_per_w = n_tokens // NW

  mesh = plsc.VectorSubcoreMesh(
      core_axis_name="c", subcore_axis_name="s",
      num_cores=NC, num_subcores=NS)

  @functools.partial(
      pl.kernel,
      out_type=jax.ShapeDtypeStruct((n_tokens, D), jnp.float32),
      mesh=mesh,
      scratch_types=[
          pltpu.VMEM((tok_rows_per_w, L), jnp.int32),
          pltpu.VMEM((CHUNK, D), jnp.float32),
          pltpu.SemaphoreType.DMA,
      ],
      compiler_params=pltpu.CompilerParams(use_tc_tiling_on_sc=False),
  )
  def embed(tok_hbm, w_hbm, out_hbm, idx_v, rows_v, gsem):
    wid = lax.axis_index("s") * NC + lax.axis_index("c")
    pltpu.sync_copy(tok_hbm.at[pl.ds(wid * tok_rows_per_w, tok_rows_per_w)],
                    idx_v)
    out_base = wid * tok_per_w

    @pl.loop(0, n_groups)
    def group_loop(g):
      cps = []
      for j in range(GROUP):
        cps.append(pltpu.async_copy(
            w_hbm.at[idx_v.at[g * GROUP + j]],
            rows_v.at[pl.ds(j * L, L)], gsem))
      for cp in cps:
        cp.wait()
      pltpu.sync_copy(rows_v, out_hbm.at[pl.ds(out_base + g * CHUNK, CHUNK)])

  return embed


def kernel(token_ids, weight):
  b, s = token_ids.shape
  n = b * s
  tok = token_ids.reshape(n // L, L).astype(jnp.int32)
  out = _build(n)(tok, weight)
  return out.reshape(b, s, D)


# speedup vs baseline: 1.4933x; 1.0116x over previous
import functools

import jax
import jax.numpy as jnp
from jax import lax
from jax.experimental import pallas as pl
from jax.experimental.pallas import tpu as pltpu
from jax.experimental.pallas import tpu_sc as plsc

NC = 2
NS = 16
NW = NC * NS

D = 32
L = 128
GROUP = 10
CHUNK = L * GROUP


def _build(n_tokens):
  assert n_tokens % (NW * 2 * CHUNK) == 0
  tok_rows_per_w = n_tokens // (NW * L)
  n_groups = tok_rows_per_w // GROUP
  tok_per_w = n_tokens // NW

  mesh = plsc.VectorSubcoreMesh(
      core_axis_name="c", subcore_axis_name="s",
      num_cores=NC, num_subcores=NS)

  @functools.partial(
      pl.kernel,
      out_type=jax.ShapeDtypeStruct((n_tokens, D), jnp.float32),
      mesh=mesh,
      scratch_types=[
          pltpu.VMEM((tok_rows_per_w, L), jnp.int32),
          pltpu.VMEM((2, CHUNK, D), jnp.float32),
          pltpu.SemaphoreType.DMA((2,)),
          pltpu.SemaphoreType.DMA((2,)),
      ],
      compiler_params=pltpu.CompilerParams(use_tc_tiling_on_sc=False),
  )
  def embed(tok_hbm, w_hbm, out_hbm, idx_v, rows_v, gsem, wsem):
    wid = lax.axis_index("s") * NC + lax.axis_index("c")
    pltpu.sync_copy(tok_hbm.at[pl.ds(wid * tok_rows_per_w, tok_rows_per_w)],
                    idx_v)
    out_base = wid * tok_per_w

    def half(k, g, b):
      @pl.when(k > 0)
      def _():
        pltpu.make_async_copy(
            rows_v.at[b], out_hbm.at[pl.ds(out_base, CHUNK)],
            wsem.at[b]).wait()
      cps = []
      for j in range(GROUP):
        cps.append(pltpu.async_copy(
            w_hbm.at[idx_v.at[g * GROUP + j]],
            rows_v.at[b].at[pl.ds(j * L, L)], gsem.at[b]))
      for cp in cps:
        cp.wait()
      pltpu.async_copy(rows_v.at[b],
                       out_hbm.at[pl.ds(out_base + g * CHUNK, CHUNK)],
                       wsem.at[b])

    @pl.loop(0, n_groups // 2)
    def group_loop(k):
      half(k, 2 * k, 0)
      half(k, 2 * k + 1, 1)

    for b in range(2):
      pltpu.make_async_copy(
          rows_v.at[b], out_hbm.at[pl.ds(out_base, CHUNK)],
          wsem.at[b]).wait()

  return embed


def kernel(token_ids, weight):
  b, s = token_ids.shape
  n = b * s
  tok = token_ids.reshape(n // L, L).astype(jnp.int32)
  out = _build(n)(tok, weight)
  return out.reshape(b, s, D)
